# Initial kernel scaffold; baseline (speedup 1.0000x reference)
import jax
import jax.numpy as jnp
from jax.experimental import pallas as pl


def kernel(input_ids, table, w_proj):
    raise NotImplementedError("write your pallas kernel here")


# speedup vs baseline: 4.1852x; 4.1852x over previous
import functools

import jax
import jax.numpy as jnp
from jax import lax
from jax.experimental import pallas as pl
from jax.experimental.pallas import tpu as pltpu
from jax.experimental.pallas import tpu_sc as plsc

_BUCKETS = 1000000
_DIM = 64
_MODEL_DIM = 1024
_N = 16384
_NC, _NS = 2, 16
_NW = _NC * _NS
_BPW = _N // _NW
_CH = 128
_NCH = _BPW // _CH

_mesh = plsc.VectorSubcoreMesh(core_axis_name="c", subcore_axis_name="s")


@functools.partial(
    pl.kernel,
    mesh=_mesh,
    compiler_params=pltpu.CompilerParams(use_tc_tiling_on_sc=False),
    out_type=jax.ShapeDtypeStruct((_N, _DIM), jnp.float32),
    scratch_types=[
        pltpu.VMEM((_BPW,), jnp.int32),
        pltpu.VMEM((_BPW,), jnp.int32),
        pltpu.VMEM((_BPW,), jnp.int32),
        pltpu.VMEM((_BPW, _DIM), jnp.float32),
        pltpu.SemaphoreType.DMA,
    ],
)
def _hash_gather(ids_hbm, prev_hbm, table_hbm, out_hbm,
                 ids_v, prev_v, idx_v, rows_v, sem):
    wid = (lax.axis_index("s") * jnp.int32(_NC)
           + lax.axis_index("c")).astype(jnp.int32)
    base = wid * jnp.int32(_BPW)
    pltpu.sync_copy(ids_hbm.at[pl.ds(base, _BPW)], ids_v)
    pltpu.sync_copy(prev_hbm.at[pl.ds(base, _BPW)], prev_v)
    for i in range(_BPW // 16):
        x = ids_v[pl.ds(i * 16, 16)]
        p = prev_v[pl.ds(i * 16, 16)]
        t = ((p >> jnp.int32(10)) * jnp.int32(48704)
             + (p & jnp.int32(1023)) * jnp.int32(92821) + x)
        idx_v[pl.ds(i * 16, 16)] = t % jnp.int32(_BUCKETS)
    copies = [
        pltpu.async_copy(
            table_hbm.at[idx_v.at[pl.ds(j * _CH, _CH)]],
            rows_v.at[pl.ds(j * _CH, _CH)],
            sem,
        )
        for j in range(_NCH)
    ]
    for c in copies:
        c.wait()
    pltpu.sync_copy(rows_v, out_hbm.at[pl.ds(base, _BPW)])


def _mm_body(x_ref, w_ref, o_ref):
    o_ref[...] = lax.dot_general(
        x_ref[...], w_ref[...], (((1,), (1,)), ((), ())),
        preferred_element_type=jnp.float32)


_MB = 1024

_mm = pl.pallas_call(
    _mm_body,
    grid=(_N // _MB,),
    in_specs=[
        pl.BlockSpec((_MB, _DIM), lambda i: (i, jnp.int32(0))),
        pl.BlockSpec((_MODEL_DIM, _DIM),
                     lambda i: (jnp.int32(0), jnp.int32(0))),
    ],
    out_specs=pl.BlockSpec((_MB, _MODEL_DIM), lambda i: (i, jnp.int32(0))),
    out_shape=jax.ShapeDtypeStruct((_N, _MODEL_DIM), jnp.float32),
)


def kernel(input_ids, table, w_proj):
    b, s = input_ids.shape
    ids = input_ids.astype(jnp.int32)
    prev = jnp.pad(ids[:, :-1], ((0, 0), (1, 0)))
    emb = _hash_gather(ids.reshape(-1), prev.reshape(-1),
                       table.astype(jnp.float32))
    out = _mm(emb, w_proj.astype(jnp.float32))
    return out.reshape(b, s, _MODEL_DIM).astype(w_proj.dtype)
